# baseline (speedup 1.0000x reference)
import functools

import jax
import jax.numpy as jnp
from jax.experimental import pallas as pl
from jax.experimental.pallas import tpu as pltpu

GRID_H, GRID_W, INPUT_DIM = 32, 32, 256
K = GRID_H * GRID_W
BLOCK_B = 1024
SLAB_K = 256
SUB = 8
N_SLABS = K // SLAB_K


def _bmu_block_kernel(x_ref, w_ref, idx_ref, qerr_ref,
                      wsq_ref, m2x_ref, xsq_ref, best_ref, bestrow_ref):
    s = pl.program_id(1)
    w = w_ref[...]

    @pl.when(pl.program_id(0) == 0)
    def _():
        wsq_ref[pl.ds(s * SLAB_K, SLAB_K), :] = (
            jnp.sum(w * w, axis=1, keepdims=True))

    @pl.when(s == 0)
    def _():
        x = x_ref[...]
        m2x_ref[...] = -2.0 * x
        ones_d = jnp.ones((1, INPUT_DIM), jnp.float32)
        xsq_ref[...] = jax.lax.dot_general(
            ones_d, x * x,
            dimension_numbers=(((1,), (1,)), ((), ())),
            preferred_element_type=jnp.float32,
        )
        best_ref[...] = jnp.full((SUB, BLOCK_B), jnp.inf, jnp.float32)
        bestrow_ref[...] = jnp.zeros((SUB, BLOCK_B), jnp.int32)

    cross = jax.lax.dot_general(
        w, m2x_ref[...],
        dimension_numbers=(((1,), (1,)), ((), ())),
        preferred_element_type=jnp.float32,
    )
    x_sq = xsq_ref[...]
    wsq = wsq_ref[pl.ds(s * SLAB_K, SLAB_K), :]

    best = best_ref[...]
    bestrow = bestrow_ref[...]
    base = s * (SLAB_K // SUB)
    for c in range(SLAB_K // SUB):
        d = (x_sq + cross[c * SUB:(c + 1) * SUB]) + wsq[c * SUB:(c + 1) * SUB]
        m = d < best
        best = jnp.minimum(best, d)
        bestrow = jnp.where(m, base + c, bestrow)
    best_ref[...] = best
    bestrow_ref[...] = bestrow

    @pl.when(s == N_SLABS - 1)
    def _():
        k = (bestrow * SUB +
             jax.lax.broadcasted_iota(jnp.int32, bestrow.shape, 0))
        minv = jnp.min(best, axis=0, keepdims=True)
        idx_ref[...] = jnp.min(jnp.where(best == minv, k, K), axis=0)
        qerr_ref[...] = jnp.sqrt(jnp.maximum(minv[0], 0.0))


@functools.partial(jax.jit)
def _bmu_search(inputs, flat_weights):
    batch = inputs.shape[0]
    grid = (batch // BLOCK_B, N_SLABS)
    return pl.pallas_call(
        _bmu_block_kernel,
        grid=grid,
        in_specs=[
            pl.BlockSpec((BLOCK_B, INPUT_DIM), lambda i, s: (i, 0)),
            pl.BlockSpec((SLAB_K, INPUT_DIM), lambda i, s: (s, 0)),
        ],
        out_specs=[
            pl.BlockSpec((BLOCK_B,), lambda i, s: (i,)),
            pl.BlockSpec((BLOCK_B,), lambda i, s: (i,)),
        ],
        out_shape=[
            jax.ShapeDtypeStruct((batch,), jnp.int32),
            jax.ShapeDtypeStruct((batch,), jnp.float32),
        ],
        scratch_shapes=[
            pltpu.VMEM((K, 1), jnp.float32),
            pltpu.VMEM((BLOCK_B, INPUT_DIM), jnp.float32),
            pltpu.VMEM((1, BLOCK_B), jnp.float32),
            pltpu.VMEM((SUB, BLOCK_B), jnp.float32),
            pltpu.VMEM((SUB, BLOCK_B), jnp.int32),
        ],
    )(inputs, flat_weights)


def kernel(inputs, weights_map):
    flat_weights = jnp.reshape(weights_map, (K, INPUT_DIM))
    idx, qerr = _bmu_search(inputs, flat_weights)
    bmu_y = idx // GRID_W
    bmu_x = idx % GRID_W
    bmu_indices = jnp.stack([bmu_y, bmu_x], axis=1)
    return bmu_indices, qerr
